# Initial kernel scaffold; baseline (speedup 1.0000x reference)
import jax
import jax.numpy as jnp
from jax.experimental import pallas as pl


def kernel(x, edge_index, edge_weight, W_enc, b_enc, S_raw, W_dec):
    raise NotImplementedError("write your pallas kernel here")


# speedup vs baseline: 1.8745x; 1.8745x over previous
import functools

import jax
import jax.numpy as jnp
from jax import lax
from jax.experimental import pallas as pl
from jax.experimental.pallas import tpu as pltpu
from jax.experimental.pallas import tpu_sc as plsc

N = 10000
E = 320000
D = 128
MAX_ITER = 10
NEUMANN_K = 10
ALPHA = 1.0
MONOTONE_M = 0.1

NC = 2
NS = 16
NT = NC * NS
ND = 320
NPAD = NT * ND
K = 128
ECAP = E + NT * K

def _wid():
    return lax.axis_index("s") * NC + lax.axis_index("c")


def _sload(ref, idx):
    return ref[pl.ds(idx, 16)][0]


def _sc_deg_body(pew_hbm, pdl_hbm, meta_hbm, deg_hbm, deg_v, wv, dlv, meta_v):
    wid = _wid()
    pltpu.sync_copy(meta_hbm, meta_v)
    poff_t = _sload(meta_v, wid)
    nch_t = _sload(meta_v, NT + wid)
    lane0 = lax.broadcasted_iota(jnp.int32, (16,), 0) == 0

    def _zero(i, _):
        deg_v[pl.ds(i * 16, 16)] = jnp.zeros((16,), jnp.float32)
        return 0

    lax.fori_loop(0, (ND + 16) // 16, _zero, 0)

    def _chunk(c, _):
        base = pl.multiple_of(poff_t + c * K, K)
        pltpu.sync_copy(pew_hbm.at[pl.ds(base, K)], wv.at[pl.ds(0, K)])
        pltpu.sync_copy(pdl_hbm.at[pl.ds(base, K)], dlv.at[pl.ds(0, K)])

        def _edge(e, _):
            dl = _sload(dlv, e)
            w_e = _sload(wv, e)
            plsc.addupdate(deg_v.at[pl.ds(dl, 16)],
                           jnp.where(lane0, w_e, 0.0))
            return 0

        lax.fori_loop(0, K, _edge, 0)
        return 0

    lax.fori_loop(0, nch_t, _chunk, 0)
    pltpu.sync_copy(deg_v.at[pl.ds(0, ND)], deg_hbm.at[pl.ds(pl.multiple_of(wid * ND, ND), ND)])


def _sc_w_body(psrc_hbm, pdl_hbm, pew_hbm, dsr_hbm, meta_hbm, pw_hbm,
               dsr_v, wv, sv, dlv, ov, meta_v):
    wid = _wid()
    pltpu.sync_copy(meta_hbm, meta_v)
    pltpu.sync_copy(dsr_hbm, dsr_v)
    base_node = wid * ND
    poff_t = _sload(meta_v, wid)
    nch_t = _sload(meta_v, NT + wid)

    def _chunk(c, _):
        base = pl.multiple_of(poff_t + c * K, K)
        pltpu.sync_copy(pew_hbm.at[pl.ds(base, K)], wv)
        pltpu.sync_copy(psrc_hbm.at[pl.ds(base, K)], sv)
        pltpu.sync_copy(pdl_hbm.at[pl.ds(base, K)], dlv)
        for j in range(K // 16):
            sl = pl.ds(j * 16, 16)
            si = sv[sl]
            di = dlv[sl] + base_node
            a = plsc.load_gather(dsr_v, [si])
            b = plsc.load_gather(dsr_v, [di])
            ov[sl] = wv[sl] * a * b
        pltpu.sync_copy(ov, pw_hbm.at[pl.ds(base, K)])
        return 0

    lax.fori_loop(0, nch_t, _chunk, 0)


def _sc_spmm_body(z_hbm, psrc_hbm, pw_hbm, pdl_hbm, meta_hbm, agg_hbm,
                  agg_v, rows_v, sv, wv, dlv, meta_v, sem):
    wid = _wid()
    pltpu.sync_copy(meta_hbm, meta_v)
    poff_t = _sload(meta_v, wid)
    nch_t = _sload(meta_v, NT + wid)

    def _zero(i, _):
        agg_v[pl.ds(i * 16, 16)] = jnp.zeros((16,), jnp.float32)
        return 0

    lax.fori_loop(0, ND * D // 16, _zero, 0)

    def _chunk(c, _):
        base = pl.multiple_of(poff_t + c * K, K)
        pltpu.sync_copy(psrc_hbm.at[pl.ds(base, K)], sv)
        pltpu.sync_copy(pw_hbm.at[pl.ds(base, K)], wv.at[pl.ds(0, K)])
        pltpu.sync_copy(pdl_hbm.at[pl.ds(base, K)], dlv.at[pl.ds(0, K)])
        pltpu.async_copy(z_hbm.at[sv], rows_v, sem).wait()

        def _edge(e, _):
            w_e = _sload(wv, e)
            dl = _sload(dlv, e)
            abase = dl * D
            for j in range(D // 16):
                v = rows_v[e, pl.ds(j * 16, 16)]
                plsc.addupdate(agg_v.at[pl.ds(abase + j * 16, 16)], v * w_e)
            return 0

        lax.fori_loop(0, K, _edge, 0)
        return 0

    lax.fori_loop(0, nch_t, _chunk, 0)
    pltpu.sync_copy(agg_v, agg_hbm.at[pl.ds(pl.multiple_of(wid * ND * D, ND * D), ND * D)])


@functools.lru_cache(maxsize=1)
def _sc_kernels():
    mesh = plsc.VectorSubcoreMesh(
        core_axis_name="c", subcore_axis_name="s",
        num_cores=NC, num_subcores=NS)
    sc_deg = pl.kernel(
        _sc_deg_body,
        out_type=jax.ShapeDtypeStruct((NPAD,), jnp.float32),
        mesh=mesh,
        scratch_types=[
            pltpu.VMEM((ND + 16,), jnp.float32),
            pltpu.VMEM((K + 16,), jnp.float32),
            pltpu.VMEM((K + 16,), jnp.int32),
            pltpu.VMEM((80,), jnp.int32),
        ],
    )
    sc_w = pl.kernel(
        _sc_w_body,
        out_type=jax.ShapeDtypeStruct((ECAP,), jnp.float32),
        mesh=mesh,
        compiler_params=pltpu.CompilerParams(needs_layout_passes=False),
        scratch_types=[
            pltpu.VMEM((NPAD,), jnp.float32),
            pltpu.VMEM((K,), jnp.float32),
            pltpu.VMEM((K,), jnp.int32),
            pltpu.VMEM((K,), jnp.int32),
            pltpu.VMEM((K,), jnp.float32),
            pltpu.VMEM((80,), jnp.int32),
        ],
    )
    sc_spmm = pl.kernel(
        _sc_spmm_body,
        out_type=jax.ShapeDtypeStruct((NPAD * D,), jnp.float32),
        mesh=mesh,
        scratch_types=[
            pltpu.VMEM((ND * D,), jnp.float32),
            pltpu.VMEM((K, D), jnp.float32),
            pltpu.VMEM((K,), jnp.int32),
            pltpu.VMEM((K + 16,), jnp.float32),
            pltpu.VMEM((K + 16,), jnp.int32),
            pltpu.VMEM((80,), jnp.int32),
            pltpu.SemaphoreType.DMA,
        ],
    )
    return sc_deg, sc_w, sc_spmm


_RB = 1280
_GRID = NPAD // _RB


def _rows_spec(blk=_RB):
    return pl.BlockSpec((blk, D), lambda i: (i, 0))


def _full_spec():
    return pl.BlockSpec((D, D), lambda i: (0, 0))


def _dotT(a, b):
    return lax.dot_general(a, b, (((1,), (1,)), ((), ())),
                           preferred_element_type=jnp.float32)


def _rsqrt_body(deg_ref, out_ref):
    out_ref[...] = lax.rsqrt(deg_ref[...] + 1e-6)


def _tc_rsqrt(deg2d):
    return pl.pallas_call(
        _rsqrt_body,
        out_shape=jax.ShapeDtypeStruct(deg2d.shape, jnp.float32),
    )(deg2d)


def _omega_body(sraw_ref, ot_ref):
    sraw = sraw_ref[...]
    eye = jnp.eye(D, dtype=jnp.float32)
    srt = lax.dot_general(sraw, eye, (((0,), (0,)), ((), ())),
                          preferred_element_type=jnp.float32)
    s = 0.5 * (sraw - srt)
    s2 = jnp.dot(s, s, preferred_element_type=jnp.float32)
    a = eye - s2
    rho = jnp.max(jnp.sum(jnp.abs(a), axis=1))
    x = eye / rho

    for _ in range(18):
        ax = jnp.dot(a, x, preferred_element_type=jnp.float32)
        x = jnp.dot(x, 2.0 * eye - ax, preferred_element_type=jnp.float32)
    ot_ref[...] = (1.0 - MONOTONE_M) * jnp.dot(
        eye + 2.0 * s + s2, x, preferred_element_type=jnp.float32)


def _tc_omega(s_raw):
    return pl.pallas_call(
        _omega_body,
        out_shape=jax.ShapeDtypeStruct((D, D), jnp.float32),
    )(s_raw)


def _encode_body(x_ref, wenc_ref, benc_ref, b_ref):
    h = _dotT(x_ref[...], wenc_ref[...]) + benc_ref[...][None, :]
    b_ref[...] = ALPHA * h


def _tc_encode(xp, w_enc, b_enc):
    return pl.pallas_call(
        _encode_body,
        grid=(_GRID,),
        in_specs=[_rows_spec(), _full_spec(),
                  pl.BlockSpec((D,), lambda i: (0,))],
        out_specs=_rows_spec(),
        out_shape=jax.ShapeDtypeStruct((NPAD, D), jnp.float32),
    )(xp, w_enc, b_enc)


def _step_body(agg_ref, ot_ref, acc_ref, term_ref, acc2_ref):
    r = ALPHA / (1.0 + ALPHA)
    t = r * jnp.dot(agg_ref[...], ot_ref[...],
                    preferred_element_type=jnp.float32)
    term_ref[...] = t
    acc2_ref[...] = acc_ref[...] + t


def _tc_step(agg, ot, acc):
    return pl.pallas_call(
        _step_body,
        grid=(_GRID,),
        in_specs=[_rows_spec(), _full_spec(), _rows_spec()],
        out_specs=[_rows_spec(), _rows_spec()],
        out_shape=[jax.ShapeDtypeStruct((NPAD, D), jnp.float32),
                   jax.ShapeDtypeStruct((NPAD, D), jnp.float32)],
    )(agg, ot, acc)


def _pre_body(z_ref, u_ref, b_ref, uh_ref, u0_ref):
    uh = 2.0 * z_ref[...] - u_ref[...]
    uh_ref[...] = uh
    u0_ref[...] = uh + b_ref[...]


def _tc_pre(z, u, binj):
    return pl.pallas_call(
        _pre_body,
        grid=(_GRID,),
        in_specs=[_rows_spec(), _rows_spec(), _rows_spec()],
        out_specs=[_rows_spec(), _rows_spec()],
        out_shape=[jax.ShapeDtypeStruct((NPAD, D), jnp.float32),
                   jax.ShapeDtypeStruct((NPAD, D), jnp.float32)],
    )(z, u, binj)


def _post_body(acc_ref, uh_ref, z_ref, u_ref):
    zh = acc_ref[...] / (1.0 + ALPHA)
    u = 2.0 * zh - uh_ref[...]
    u_ref[...] = u
    z_ref[...] = jnp.maximum(u, 0.0)


def _tc_post(acc, uh):
    return pl.pallas_call(
        _post_body,
        grid=(_GRID,),
        in_specs=[_rows_spec(), _rows_spec()],
        out_specs=[_rows_spec(), _rows_spec()],
        out_shape=[jax.ShapeDtypeStruct((NPAD, D), jnp.float32),
                   jax.ShapeDtypeStruct((NPAD, D), jnp.float32)],
    )(acc, uh)


def _decode_body(z_ref, wdec_ref, out_ref):
    out_ref[...] = _dotT(jnp.maximum(z_ref[...], 0.0), wdec_ref[...])


def _tc_decode(z, w_dec):
    return pl.pallas_call(
        _decode_body,
        grid=(_GRID,),
        in_specs=[_rows_spec(), _full_spec()],
        out_specs=_rows_spec(),
        out_shape=jax.ShapeDtypeStruct((NPAD, D), jnp.float32),
    )(z, w_dec)


def kernel(x, edge_index, edge_weight, W_enc, b_enc, S_raw, W_dec):
    src, dst, ew = edge_index[0], edge_index[1], edge_weight
    order = jnp.argsort(dst)
    src_s = src[order]
    dst_s = dst[order]
    ew_s = ew[order]
    tile = dst_s // ND
    off = jnp.searchsorted(dst_s, (jnp.arange(NT + 1) * ND).astype(jnp.int32))
    cnt = (off[1:] - off[:-1]).astype(jnp.int32)
    pcnt = ((cnt + K - 1) // K) * K
    poff = jnp.concatenate([jnp.zeros((1,), jnp.int32),
                            jnp.cumsum(pcnt)[:-1].astype(jnp.int32)])
    pos = poff[tile] + (jnp.arange(E, dtype=jnp.int32) - off[tile].astype(jnp.int32))
    psrc = jnp.zeros((ECAP,), jnp.int32).at[pos].set(src_s)
    pdl = jnp.zeros((ECAP,), jnp.int32).at[pos].set(
        (dst_s - tile * ND).astype(jnp.int32))
    pew = jnp.zeros((ECAP,), jnp.float32).at[pos].set(ew_s)
    meta = jnp.concatenate([poff, (pcnt // K).astype(jnp.int32)])
    meta = jnp.pad(meta, (0, 80 - 2 * NT))
    xp = jnp.pad(x, ((0, NPAD - N), (0, 0)))

    sc_deg, sc_w, sc_spmm = _sc_kernels()
    deg = sc_deg(pew, pdl, meta)
    dsr = _tc_rsqrt(deg.reshape(NPAD // D, D)).reshape(NPAD)
    pw = sc_w(psrc, pdl, pew, dsr, meta)
    ot = _tc_omega(S_raw)
    binj = _tc_encode(xp, W_enc, b_enc)

    z0 = jnp.zeros((NPAD, D), jnp.float32)

    def pr_body(i, zu):
        z, u = zu
        uh, u0 = _tc_pre(z, u, binj)

        def neu_body(k, ta):
            term, acc = ta
            agg = sc_spmm(term, psrc, pw, pdl, meta).reshape(NPAD, D)
            return _tc_step(agg, ot, acc)

        _, acc = lax.fori_loop(0, NEUMANN_K, neu_body, (u0, u0))
        return _tc_post(acc, uh)

    z, _ = lax.fori_loop(0, MAX_ITER, pr_body, (z0, z0))
    out = _tc_decode(z, W_dec)
    return out[:N]


# speedup vs baseline: 2.0211x; 1.0782x over previous
import functools

import jax
import jax.numpy as jnp
from jax import lax
from jax.experimental import pallas as pl
from jax.experimental.pallas import tpu as pltpu
from jax.experimental.pallas import tpu_sc as plsc

N = 10000
E = 320000
D = 128
MAX_ITER = 10
NEUMANN_K = 10
ALPHA = 1.0
MONOTONE_M = 0.1

NC = 2
NS = 16
NT = NC * NS
ND = 320
NPAD = NT * ND
K = 128
ECAP = E + NT * 2 * K

def _wid():
    return lax.axis_index("s") * NC + lax.axis_index("c")


def _sload(ref, idx):
    return ref[pl.ds(idx, 16)][0]


def _sc_deg_body(pew_hbm, pdl_hbm, meta_hbm, deg_hbm, deg_v, wv, dlv, meta_v):
    wid = _wid()
    pltpu.sync_copy(meta_hbm, meta_v)
    poff_t = _sload(meta_v, wid)
    nch_t = _sload(meta_v, NT + wid)
    lane0 = lax.broadcasted_iota(jnp.int32, (16,), 0) == 0

    def _zero(i, _):
        deg_v[pl.ds(i * 16, 16)] = jnp.zeros((16,), jnp.float32)
        return 0

    lax.fori_loop(0, (ND + 16) // 16, _zero, 0)

    def _chunk(c, _):
        base = pl.multiple_of(poff_t + c * K, K)
        pltpu.sync_copy(pew_hbm.at[pl.ds(base, K)], wv.at[pl.ds(0, K)])
        pltpu.sync_copy(pdl_hbm.at[pl.ds(base, K)], dlv.at[pl.ds(0, K)])

        def _edge(e, _):
            dl = _sload(dlv, e)
            w_e = _sload(wv, e)
            plsc.addupdate(deg_v.at[pl.ds(dl, 16)],
                           jnp.where(lane0, w_e, 0.0))
            return 0

        lax.fori_loop(0, K, _edge, 0)
        return 0

    lax.fori_loop(0, nch_t, _chunk, 0)
    pltpu.sync_copy(deg_v.at[pl.ds(0, ND)], deg_hbm.at[pl.ds(pl.multiple_of(wid * ND, ND), ND)])


def _sc_w_body(psrc_hbm, pdl_hbm, pew_hbm, dsr_hbm, meta_hbm, pw_hbm,
               dsr_v, wv, sv, dlv, ov, meta_v):
    wid = _wid()
    pltpu.sync_copy(meta_hbm, meta_v)
    pltpu.sync_copy(dsr_hbm, dsr_v)
    base_node = wid * ND
    poff_t = _sload(meta_v, wid)
    nch_t = _sload(meta_v, NT + wid)

    def _chunk(c, _):
        base = pl.multiple_of(poff_t + c * K, K)
        pltpu.sync_copy(pew_hbm.at[pl.ds(base, K)], wv)
        pltpu.sync_copy(psrc_hbm.at[pl.ds(base, K)], sv)
        pltpu.sync_copy(pdl_hbm.at[pl.ds(base, K)], dlv)
        for j in range(K // 16):
            sl = pl.ds(j * 16, 16)
            si = sv[sl]
            di = dlv[sl] + base_node
            a = plsc.load_gather(dsr_v, [si])
            b = plsc.load_gather(dsr_v, [di])
            ov[sl] = wv[sl] * a * b
        pltpu.sync_copy(ov, pw_hbm.at[pl.ds(base, K)])
        return 0

    lax.fori_loop(0, nch_t, _chunk, 0)


def _sc_spmm_body(z_hbm, psrc_hbm, pdw_hbm, meta_hbm, agg_hbm,
                  agg_v, rows_a, rows_b, sv_a, sv_b, pdw_a, pdw_b,
                  meta_v, sem_a, sem_b):
    wid = _wid()
    pltpu.sync_copy(meta_hbm, meta_v)
    poff_t = _sload(meta_v, wid)
    nch_t = _sload(meta_v, NT + wid)
    nch2 = nch_t // 2

    def _zero(i, _):
        agg_v[pl.ds(i * 16, 16)] = jnp.zeros((16,), jnp.float32)
        return 0

    lax.fori_loop(0, ND * D // 16, _zero, 0)

    def _fetch_meta(c, sv, pdw):
        base = pl.multiple_of(poff_t + c * K, K)
        pltpu.sync_copy(psrc_hbm.at[pl.ds(base, K)], sv)
        pltpu.sync_copy(pdw_hbm.at[pl.ds(2 * base, 2 * K)],
                        pdw.at[pl.ds(0, 2 * K)])

    def _accum(rows, pdw):
        def _edge(e, _):
            dl = _sload(pdw, e)
            w_e = plsc.bitcast(pdw[pl.ds(K + e, 16)], jnp.float32)[0]
            abase = dl * D
            for j in range(D // 16):
                v = rows[e, pl.ds(j * 16, 16)]
                plsc.addupdate(agg_v.at[pl.ds(abase + j * 16, 16)], v * w_e)
            return 0

        lax.fori_loop(0, K, _edge, 0, unroll=2)

    @pl.when(nch2 > 0)
    def _run():
        _fetch_meta(0, sv_a, pdw_a)
        pltpu.async_copy(z_hbm.at[sv_a], rows_a, sem_a)

        def _pair(c2, _):
            c1 = 2 * c2 + 1
            _fetch_meta(c1, sv_b, pdw_b)
            pltpu.async_copy(z_hbm.at[sv_b], rows_b, sem_b)
            pltpu.make_async_copy(z_hbm.at[sv_a], rows_a, sem_a).wait()
            _accum(rows_a, pdw_a)
            cn = jnp.minimum(2 * c2 + 2, nch_t - 2)
            _fetch_meta(cn, sv_a, pdw_a)
            pltpu.async_copy(z_hbm.at[sv_a], rows_a, sem_a)
            pltpu.make_async_copy(z_hbm.at[sv_b], rows_b, sem_b).wait()
            _accum(rows_b, pdw_b)
            return 0

        lax.fori_loop(0, nch2, _pair, 0)
        pltpu.make_async_copy(z_hbm.at[sv_a], rows_a, sem_a).wait()

    pltpu.sync_copy(agg_v, agg_hbm.at[pl.ds(pl.multiple_of(wid * ND * D, ND * D), ND * D)])


@functools.lru_cache(maxsize=1)
def _sc_kernels():
    mesh = plsc.VectorSubcoreMesh(
        core_axis_name="c", subcore_axis_name="s",
        num_cores=NC, num_subcores=NS)
    sc_deg = pl.kernel(
        _sc_deg_body,
        out_type=jax.ShapeDtypeStruct((NPAD,), jnp.float32),
        mesh=mesh,
        scratch_types=[
            pltpu.VMEM((ND + 16,), jnp.float32),
            pltpu.VMEM((K + 16,), jnp.float32),
            pltpu.VMEM((K + 16,), jnp.int32),
            pltpu.VMEM((80,), jnp.int32),
        ],
    )
    sc_w = pl.kernel(
        _sc_w_body,
        out_type=jax.ShapeDtypeStruct((ECAP,), jnp.float32),
        mesh=mesh,
        compiler_params=pltpu.CompilerParams(needs_layout_passes=False),
        scratch_types=[
            pltpu.VMEM((NPAD,), jnp.float32),
            pltpu.VMEM((K,), jnp.float32),
            pltpu.VMEM((K,), jnp.int32),
            pltpu.VMEM((K,), jnp.int32),
            pltpu.VMEM((K,), jnp.float32),
            pltpu.VMEM((80,), jnp.int32),
        ],
    )
    sc_spmm = pl.kernel(
        _sc_spmm_body,
        out_type=jax.ShapeDtypeStruct((NPAD * D,), jnp.float32),
        mesh=mesh,
        compiler_params=pltpu.CompilerParams(needs_layout_passes=False),
        scratch_types=[
            pltpu.VMEM((ND * D,), jnp.float32),
            pltpu.VMEM((K, D), jnp.float32),
            pltpu.VMEM((K, D), jnp.float32),
            pltpu.VMEM((K,), jnp.int32),
            pltpu.VMEM((K,), jnp.int32),
            pltpu.VMEM((2 * K + 16,), jnp.int32),
            pltpu.VMEM((2 * K + 16,), jnp.int32),
            pltpu.VMEM((80,), jnp.int32),
            pltpu.SemaphoreType.DMA,
            pltpu.SemaphoreType.DMA,
        ],
    )
    return sc_deg, sc_w, sc_spmm


_RB = 1280
_GRID = NPAD // _RB


def _rows_spec(blk=_RB):
    return pl.BlockSpec((blk, D), lambda i: (i, 0))


def _full_spec():
    return pl.BlockSpec((D, D), lambda i: (0, 0))


def _dotT(a, b):
    return lax.dot_general(a, b, (((1,), (1,)), ((), ())),
                           preferred_element_type=jnp.float32)


def _rsqrt_body(deg_ref, out_ref):
    out_ref[...] = lax.rsqrt(deg_ref[...] + 1e-6)


def _tc_rsqrt(deg2d):
    return pl.pallas_call(
        _rsqrt_body,
        out_shape=jax.ShapeDtypeStruct(deg2d.shape, jnp.float32),
    )(deg2d)


def _omega_body(sraw_ref, ot_ref):
    sraw = sraw_ref[...]
    eye = jnp.eye(D, dtype=jnp.float32)
    srt = lax.dot_general(sraw, eye, (((0,), (0,)), ((), ())),
                          preferred_element_type=jnp.float32)
    s = 0.5 * (sraw - srt)
    s2 = jnp.dot(s, s, preferred_element_type=jnp.float32)
    a = eye - s2
    rho = jnp.max(jnp.sum(jnp.abs(a), axis=1))
    x = eye / rho

    for _ in range(18):
        ax = jnp.dot(a, x, preferred_element_type=jnp.float32)
        x = jnp.dot(x, 2.0 * eye - ax, preferred_element_type=jnp.float32)
    ot_ref[...] = (1.0 - MONOTONE_M) * jnp.dot(
        eye + 2.0 * s + s2, x, preferred_element_type=jnp.float32)


def _tc_omega(s_raw):
    return pl.pallas_call(
        _omega_body,
        out_shape=jax.ShapeDtypeStruct((D, D), jnp.float32),
    )(s_raw)


def _encode_body(x_ref, wenc_ref, benc_ref, b_ref):
    h = _dotT(x_ref[...], wenc_ref[...]) + benc_ref[...][None, :]
    b_ref[...] = ALPHA * h


def _tc_encode(xp, w_enc, b_enc):
    return pl.pallas_call(
        _encode_body,
        grid=(_GRID,),
        in_specs=[_rows_spec(), _full_spec(),
                  pl.BlockSpec((D,), lambda i: (0,))],
        out_specs=_rows_spec(),
        out_shape=jax.ShapeDtypeStruct((NPAD, D), jnp.float32),
    )(xp, w_enc, b_enc)


def _step_body(agg_ref, ot_ref, acc_ref, term_ref, acc2_ref):
    r = ALPHA / (1.0 + ALPHA)
    t = r * jnp.dot(agg_ref[...], ot_ref[...],
                    preferred_element_type=jnp.float32)
    term_ref[...] = t
    acc2_ref[...] = acc_ref[...] + t


def _tc_step(agg, ot, acc):
    return pl.pallas_call(
        _step_body,
        grid=(_GRID,),
        in_specs=[_rows_spec(), _full_spec(), _rows_spec()],
        out_specs=[_rows_spec(), _rows_spec()],
        out_shape=[jax.ShapeDtypeStruct((NPAD, D), jnp.float32),
                   jax.ShapeDtypeStruct((NPAD, D), jnp.float32)],
    )(agg, ot, acc)


def _pre_body(z_ref, u_ref, b_ref, uh_ref, u0_ref):
    uh = 2.0 * z_ref[...] - u_ref[...]
    uh_ref[...] = uh
    u0_ref[...] = uh + b_ref[...]


def _tc_pre(z, u, binj):
    return pl.pallas_call(
        _pre_body,
        grid=(_GRID,),
        in_specs=[_rows_spec(), _rows_spec(), _rows_spec()],
        out_specs=[_rows_spec(), _rows_spec()],
        out_shape=[jax.ShapeDtypeStruct((NPAD, D), jnp.float32),
                   jax.ShapeDtypeStruct((NPAD, D), jnp.float32)],
    )(z, u, binj)


def _post_body(acc_ref, uh_ref, z_ref, u_ref):
    zh = acc_ref[...] / (1.0 + ALPHA)
    u = 2.0 * zh - uh_ref[...]
    u_ref[...] = u
    z_ref[...] = jnp.maximum(u, 0.0)


def _tc_post(acc, uh):
    return pl.pallas_call(
        _post_body,
        grid=(_GRID,),
        in_specs=[_rows_spec(), _rows_spec()],
        out_specs=[_rows_spec(), _rows_spec()],
        out_shape=[jax.ShapeDtypeStruct((NPAD, D), jnp.float32),
                   jax.ShapeDtypeStruct((NPAD, D), jnp.float32)],
    )(acc, uh)


def _decode_body(z_ref, wdec_ref, out_ref):
    out_ref[...] = _dotT(jnp.maximum(z_ref[...], 0.0), wdec_ref[...])


def _tc_decode(z, w_dec):
    return pl.pallas_call(
        _decode_body,
        grid=(_GRID,),
        in_specs=[_rows_spec(), _full_spec()],
        out_specs=_rows_spec(),
        out_shape=jax.ShapeDtypeStruct((NPAD, D), jnp.float32),
    )(z, w_dec)


def kernel(x, edge_index, edge_weight, W_enc, b_enc, S_raw, W_dec):
    src, dst, ew = edge_index[0], edge_index[1], edge_weight
    order = jnp.argsort(dst)
    src_s = src[order]
    dst_s = dst[order]
    ew_s = ew[order]
    tile = dst_s // ND
    off = jnp.searchsorted(dst_s, (jnp.arange(NT + 1) * ND).astype(jnp.int32))
    cnt = (off[1:] - off[:-1]).astype(jnp.int32)
    pcnt = ((cnt + 2 * K - 1) // (2 * K)) * (2 * K)
    poff = jnp.concatenate([jnp.zeros((1,), jnp.int32),
                            jnp.cumsum(pcnt)[:-1].astype(jnp.int32)])
    pos = poff[tile] + (jnp.arange(E, dtype=jnp.int32) - off[tile].astype(jnp.int32))
    psrc = jnp.zeros((ECAP,), jnp.int32).at[pos].set(src_s)
    pdl = jnp.zeros((ECAP,), jnp.int32).at[pos].set(
        (dst_s - tile * ND).astype(jnp.int32))
    pew = jnp.zeros((ECAP,), jnp.float32).at[pos].set(ew_s)
    meta = jnp.concatenate([poff, (pcnt // K).astype(jnp.int32)])
    meta = jnp.pad(meta, (0, 80 - 2 * NT))
    xp = jnp.pad(x, ((0, NPAD - N), (0, 0)))

    sc_deg, sc_w, sc_spmm = _sc_kernels()
    deg = sc_deg(pew, pdl, meta)
    dsr = _tc_rsqrt(deg.reshape(NPAD // D, D)).reshape(NPAD)
    pw = sc_w(psrc, pdl, pew, dsr, meta)
    pdw = jnp.stack(
        [pdl.reshape(-1, K),
         lax.bitcast_convert_type(pw, jnp.int32).reshape(-1, K)],
        axis=1).reshape(2 * ECAP)
    ot = _tc_omega(S_raw)
    binj = _tc_encode(xp, W_enc, b_enc)

    z0 = jnp.zeros((NPAD, D), jnp.float32)

    def pr_body(i, zu):
        z, u = zu
        uh, u0 = _tc_pre(z, u, binj)

        def neu_body(k, ta):
            term, acc = ta
            agg = sc_spmm(term, psrc, pdw, meta).reshape(NPAD, D)
            return _tc_step(agg, ot, acc)

        _, acc = lax.fori_loop(0, NEUMANN_K, neu_body, (u0, u0))
        return _tc_post(acc, uh)

    z, _ = lax.fori_loop(0, MAX_ITER, pr_body, (z0, z0))
    out = _tc_decode(z, W_dec)
    return out[:N]


# speedup vs baseline: 2.2779x; 1.1270x over previous
import functools

import jax
import jax.numpy as jnp
from jax import lax
from jax.experimental import pallas as pl
from jax.experimental.pallas import tpu as pltpu
from jax.experimental.pallas import tpu_sc as plsc

N = 10000
E = 320000
D = 128
MAX_ITER = 10
NEUMANN_K = 10
ALPHA = 1.0
MONOTONE_M = 0.1

NC = 2
NS = 16
NT = NC * NS
ND = 320
NPAD = NT * ND
K = 128
ECAP = E + NT * 2 * K

def _wid():
    return lax.axis_index("s") * NC + lax.axis_index("c")


def _sload(ref, idx):
    return ref[pl.ds(idx, 16)][0]


def _sc_deg_body(pew_hbm, pdl_hbm, meta_hbm, deg_hbm, deg_v, wv, dlv, meta_v):
    wid = _wid()
    pltpu.sync_copy(meta_hbm, meta_v)
    poff_t = _sload(meta_v, wid)
    nch_t = _sload(meta_v, NT + wid)
    lane0 = lax.broadcasted_iota(jnp.int32, (16,), 0) == 0

    def _zero(i, _):
        deg_v[pl.ds(i * 16, 16)] = jnp.zeros((16,), jnp.float32)
        return 0

    lax.fori_loop(0, (ND + 16) // 16, _zero, 0)

    def _chunk(c, _):
        base = pl.multiple_of(poff_t + c * K, K)
        pltpu.sync_copy(pew_hbm.at[pl.ds(base, K)], wv.at[pl.ds(0, K)])
        pltpu.sync_copy(pdl_hbm.at[pl.ds(base, K)], dlv.at[pl.ds(0, K)])

        def _edge(e, _):
            dl = _sload(dlv, e)
            w_e = _sload(wv, e)
            plsc.addupdate(deg_v.at[pl.ds(dl, 16)],
                           jnp.where(lane0, w_e, 0.0))
            return 0

        lax.fori_loop(0, K, _edge, 0)
        return 0

    lax.fori_loop(0, nch_t, _chunk, 0)
    pltpu.sync_copy(deg_v.at[pl.ds(0, ND)], deg_hbm.at[pl.ds(pl.multiple_of(wid * ND, ND), ND)])


def _sc_w_body(psrc_hbm, pdl_hbm, pew_hbm, dsr_hbm, meta_hbm, pw_hbm,
               dsr_v, wv, sv, dlv, ov, meta_v):
    wid = _wid()
    pltpu.sync_copy(meta_hbm, meta_v)
    pltpu.sync_copy(dsr_hbm, dsr_v)
    base_node = wid * ND
    poff_t = _sload(meta_v, wid)
    nch_t = _sload(meta_v, NT + wid)

    def _chunk(c, _):
        base = pl.multiple_of(poff_t + c * K, K)
        pltpu.sync_copy(pew_hbm.at[pl.ds(base, K)], wv)
        pltpu.sync_copy(psrc_hbm.at[pl.ds(base, K)], sv)
        pltpu.sync_copy(pdl_hbm.at[pl.ds(base, K)], dlv)
        for j in range(K // 16):
            sl = pl.ds(j * 16, 16)
            si = sv[sl]
            di = dlv[sl] + base_node
            a = plsc.load_gather(dsr_v, [si])
            b = plsc.load_gather(dsr_v, [di])
            ov[sl] = wv[sl] * a * b
        pltpu.sync_copy(ov, pw_hbm.at[pl.ds(base, K)])
        return 0

    lax.fori_loop(0, nch_t, _chunk, 0)


def _sc_spmm_body(z_hbm, psrc_hbm, pdw_hbm, meta_hbm, agg_hbm,
                  agg_v, rows_a, rows_b, sv_a, sv_b, pdw_a, pdw_b,
                  meta_v, sem_a, sem_b):
    wid = _wid()
    pltpu.sync_copy(meta_hbm, meta_v)
    poff_t = _sload(meta_v, wid)
    nch_t = _sload(meta_v, NT + wid)
    nch2 = nch_t // 2

    def _zero(i, _):
        agg_v[pl.ds(i * 16, 16)] = jnp.zeros((16,), jnp.float32)
        return 0

    lax.fori_loop(0, ND * D // 16, _zero, 0)

    def _fetch_meta(c, sv, pdw):
        base = pl.multiple_of(poff_t + c * K, K)
        pltpu.sync_copy(psrc_hbm.at[pl.ds(base, K)], sv)
        pltpu.sync_copy(pdw_hbm.at[pl.ds(2 * base, 2 * K)],
                        pdw.at[pl.ds(0, 2 * K)])

    def _accum(rows, pdw):
        def _group(g, _):
            e0 = g * 16
            dl16 = pdw[pl.ds(e0, 16)] * D
            w16 = plsc.bitcast(pdw[pl.ds(K + e0, 16)], jnp.float32)
            for i in range(16):
                abase = dl16[i]
                w_e = w16[i]
                for j in range(D // 16):
                    v = rows[e0 + i, pl.ds(j * 16, 16)]
                    plsc.addupdate(agg_v.at[pl.ds(abase + j * 16, 16)],
                                   v * w_e)
            return 0

        lax.fori_loop(0, K // 16, _group, 0)

    @pl.when(nch2 > 0)
    def _run():
        _fetch_meta(0, sv_a, pdw_a)
        pltpu.async_copy(z_hbm.at[sv_a], rows_a, sem_a)

        def _pair(c2, _):
            c1 = 2 * c2 + 1
            _fetch_meta(c1, sv_b, pdw_b)
            pltpu.async_copy(z_hbm.at[sv_b], rows_b, sem_b)
            pltpu.make_async_copy(z_hbm.at[sv_a], rows_a, sem_a).wait()
            _accum(rows_a, pdw_a)
            cn = jnp.minimum(2 * c2 + 2, nch_t - 2)
            _fetch_meta(cn, sv_a, pdw_a)
            pltpu.async_copy(z_hbm.at[sv_a], rows_a, sem_a)
            pltpu.make_async_copy(z_hbm.at[sv_b], rows_b, sem_b).wait()
            _accum(rows_b, pdw_b)
            return 0

        lax.fori_loop(0, nch2, _pair, 0)
        pltpu.make_async_copy(z_hbm.at[sv_a], rows_a, sem_a).wait()

    pltpu.sync_copy(agg_v, agg_hbm.at[pl.ds(pl.multiple_of(wid * ND * D, ND * D), ND * D)])


@functools.lru_cache(maxsize=1)
def _sc_kernels():
    mesh = plsc.VectorSubcoreMesh(
        core_axis_name="c", subcore_axis_name="s",
        num_cores=NC, num_subcores=NS)
    sc_deg = pl.kernel(
        _sc_deg_body,
        out_type=jax.ShapeDtypeStruct((NPAD,), jnp.float32),
        mesh=mesh,
        scratch_types=[
            pltpu.VMEM((ND + 16,), jnp.float32),
            pltpu.VMEM((K + 16,), jnp.float32),
            pltpu.VMEM((K + 16,), jnp.int32),
            pltpu.VMEM((80,), jnp.int32),
        ],
    )
    sc_w = pl.kernel(
        _sc_w_body,
        out_type=jax.ShapeDtypeStruct((ECAP,), jnp.float32),
        mesh=mesh,
        compiler_params=pltpu.CompilerParams(needs_layout_passes=False),
        scratch_types=[
            pltpu.VMEM((NPAD,), jnp.float32),
            pltpu.VMEM((K,), jnp.float32),
            pltpu.VMEM((K,), jnp.int32),
            pltpu.VMEM((K,), jnp.int32),
            pltpu.VMEM((K,), jnp.float32),
            pltpu.VMEM((80,), jnp.int32),
        ],
    )
    sc_spmm = pl.kernel(
        _sc_spmm_body,
        out_type=jax.ShapeDtypeStruct((NPAD * D,), jnp.float32),
        mesh=mesh,
        compiler_params=pltpu.CompilerParams(needs_layout_passes=False),
        scratch_types=[
            pltpu.VMEM((ND * D,), jnp.float32),
            pltpu.VMEM((K, D), jnp.float32),
            pltpu.VMEM((K, D), jnp.float32),
            pltpu.VMEM((K,), jnp.int32),
            pltpu.VMEM((K,), jnp.int32),
            pltpu.VMEM((2 * K + 16,), jnp.int32),
            pltpu.VMEM((2 * K + 16,), jnp.int32),
            pltpu.VMEM((80,), jnp.int32),
            pltpu.SemaphoreType.DMA,
            pltpu.SemaphoreType.DMA,
        ],
    )
    return sc_deg, sc_w, sc_spmm


_RB = 1280
_GRID = NPAD // _RB


def _rows_spec(blk=_RB):
    return pl.BlockSpec((blk, D), lambda i: (i, 0))


def _full_spec():
    return pl.BlockSpec((D, D), lambda i: (0, 0))


def _dotT(a, b):
    return lax.dot_general(a, b, (((1,), (1,)), ((), ())),
                           preferred_element_type=jnp.float32)


def _rsqrt_body(deg_ref, out_ref):
    out_ref[...] = lax.rsqrt(deg_ref[...] + 1e-6)


def _tc_rsqrt(deg2d):
    return pl.pallas_call(
        _rsqrt_body,
        out_shape=jax.ShapeDtypeStruct(deg2d.shape, jnp.float32),
    )(deg2d)


def _omega_body(sraw_ref, ot_ref):
    sraw = sraw_ref[...]
    eye = jnp.eye(D, dtype=jnp.float32)
    srt = lax.dot_general(sraw, eye, (((0,), (0,)), ((), ())),
                          preferred_element_type=jnp.float32)
    s = 0.5 * (sraw - srt)
    s2 = jnp.dot(s, s, preferred_element_type=jnp.float32)
    a = eye - s2
    rho = jnp.max(jnp.sum(jnp.abs(a), axis=1))
    x = eye / rho

    for _ in range(18):
        ax = jnp.dot(a, x, preferred_element_type=jnp.float32)
        x = jnp.dot(x, 2.0 * eye - ax, preferred_element_type=jnp.float32)
    ot_ref[...] = (1.0 - MONOTONE_M) * jnp.dot(
        eye + 2.0 * s + s2, x, preferred_element_type=jnp.float32)


def _tc_omega(s_raw):
    return pl.pallas_call(
        _omega_body,
        out_shape=jax.ShapeDtypeStruct((D, D), jnp.float32),
    )(s_raw)


def _encode_body(x_ref, wenc_ref, benc_ref, b_ref):
    h = _dotT(x_ref[...], wenc_ref[...]) + benc_ref[...][None, :]
    b_ref[...] = ALPHA * h


def _tc_encode(xp, w_enc, b_enc):
    return pl.pallas_call(
        _encode_body,
        grid=(_GRID,),
        in_specs=[_rows_spec(), _full_spec(),
                  pl.BlockSpec((D,), lambda i: (0,))],
        out_specs=_rows_spec(),
        out_shape=jax.ShapeDtypeStruct((NPAD, D), jnp.float32),
    )(xp, w_enc, b_enc)


def _step_body(agg_ref, ot_ref, acc_ref, term_ref, acc2_ref):
    r = ALPHA / (1.0 + ALPHA)
    t = r * jnp.dot(agg_ref[...], ot_ref[...],
                    preferred_element_type=jnp.float32)
    term_ref[...] = t
    acc2_ref[...] = acc_ref[...] + t


def _tc_step(agg, ot, acc):
    return pl.pallas_call(
        _step_body,
        grid=(_GRID,),
        in_specs=[_rows_spec(), _full_spec(), _rows_spec()],
        out_specs=[_rows_spec(), _rows_spec()],
        out_shape=[jax.ShapeDtypeStruct((NPAD, D), jnp.float32),
                   jax.ShapeDtypeStruct((NPAD, D), jnp.float32)],
    )(agg, ot, acc)


def _pre_body(z_ref, u_ref, b_ref, uh_ref, u0_ref):
    uh = 2.0 * z_ref[...] - u_ref[...]
    uh_ref[...] = uh
    u0_ref[...] = uh + b_ref[...]


def _tc_pre(z, u, binj):
    return pl.pallas_call(
        _pre_body,
        grid=(_GRID,),
        in_specs=[_rows_spec(), _rows_spec(), _rows_spec()],
        out_specs=[_rows_spec(), _rows_spec()],
        out_shape=[jax.ShapeDtypeStruct((NPAD, D), jnp.float32),
                   jax.ShapeDtypeStruct((NPAD, D), jnp.float32)],
    )(z, u, binj)


def _post_body(acc_ref, uh_ref, z_ref, u_ref):
    zh = acc_ref[...] / (1.0 + ALPHA)
    u = 2.0 * zh - uh_ref[...]
    u_ref[...] = u
    z_ref[...] = jnp.maximum(u, 0.0)


def _tc_post(acc, uh):
    return pl.pallas_call(
        _post_body,
        grid=(_GRID,),
        in_specs=[_rows_spec(), _rows_spec()],
        out_specs=[_rows_spec(), _rows_spec()],
        out_shape=[jax.ShapeDtypeStruct((NPAD, D), jnp.float32),
                   jax.ShapeDtypeStruct((NPAD, D), jnp.float32)],
    )(acc, uh)


def _decode_body(z_ref, wdec_ref, out_ref):
    out_ref[...] = _dotT(jnp.maximum(z_ref[...], 0.0), wdec_ref[...])


def _tc_decode(z, w_dec):
    return pl.pallas_call(
        _decode_body,
        grid=(_GRID,),
        in_specs=[_rows_spec(), _full_spec()],
        out_specs=_rows_spec(),
        out_shape=jax.ShapeDtypeStruct((NPAD, D), jnp.float32),
    )(z, w_dec)


def kernel(x, edge_index, edge_weight, W_enc, b_enc, S_raw, W_dec):
    src, dst, ew = edge_index[0], edge_index[1], edge_weight
    order = jnp.argsort(dst)
    src_s = src[order]
    dst_s = dst[order]
    ew_s = ew[order]
    tile = dst_s // ND
    off = jnp.searchsorted(dst_s, (jnp.arange(NT + 1) * ND).astype(jnp.int32))
    cnt = (off[1:] - off[:-1]).astype(jnp.int32)
    pcnt = ((cnt + 2 * K - 1) // (2 * K)) * (2 * K)
    poff = jnp.concatenate([jnp.zeros((1,), jnp.int32),
                            jnp.cumsum(pcnt)[:-1].astype(jnp.int32)])
    pos = poff[tile] + (jnp.arange(E, dtype=jnp.int32) - off[tile].astype(jnp.int32))
    psrc = jnp.zeros((ECAP,), jnp.int32).at[pos].set(src_s)
    pdl = jnp.zeros((ECAP,), jnp.int32).at[pos].set(
        (dst_s - tile * ND).astype(jnp.int32))
    pew = jnp.zeros((ECAP,), jnp.float32).at[pos].set(ew_s)
    meta = jnp.concatenate([poff, (pcnt // K).astype(jnp.int32)])
    meta = jnp.pad(meta, (0, 80 - 2 * NT))
    xp = jnp.pad(x, ((0, NPAD - N), (0, 0)))

    sc_deg, sc_w, sc_spmm = _sc_kernels()
    deg = sc_deg(pew, pdl, meta)
    dsr = _tc_rsqrt(deg.reshape(NPAD // D, D)).reshape(NPAD)
    pw = sc_w(psrc, pdl, pew, dsr, meta)
    pdw = jnp.stack(
        [pdl.reshape(-1, K),
         lax.bitcast_convert_type(pw, jnp.int32).reshape(-1, K)],
        axis=1).reshape(2 * ECAP)
    ot = _tc_omega(S_raw)
    binj = _tc_encode(xp, W_enc, b_enc)

    z0 = jnp.zeros((NPAD, D), jnp.float32)

    def pr_body(i, zu):
        z, u = zu
        uh, u0 = _tc_pre(z, u, binj)

        def neu_body(k, ta):
            term, acc = ta
            agg = sc_spmm(term, psrc, pdw, meta).reshape(NPAD, D)
            return _tc_step(agg, ot, acc)

        _, acc = lax.fori_loop(0, NEUMANN_K, neu_body, (u0, u0))
        return _tc_post(acc, uh)

    z, _ = lax.fori_loop(0, MAX_ITER, pr_body, (z0, z0))
    out = _tc_decode(z, W_dec)
    return out[:N]
